# baseline (speedup 1.0000x reference)
import functools

import jax
import jax.numpy as jnp
from jax import lax
from jax.experimental import pallas as pl
from jax.experimental.pallas import tpu as pltpu
from jax.experimental.pallas import tpu_sc as plsc

NC = 2
NS = 16
NW = NC * NS
CH = 80


def _dot_t(a, b):
    return lax.dot_general(a, b, (((1,), (1,)), ((), ())),
                           preferred_element_type=jnp.float32)


def _uv_body(x_ref, w_ref, b_ref, u_ref, v_ref):
    d = x_ref.shape[1]
    xv = x_ref[...]
    u_ref[...] = _dot_t(xv, w_ref[:, :d]) + b_ref[...]
    v_ref[...] = _dot_t(xv, w_ref[:, d:])


def _edge_body(s_ref, w_ref, b_ref, m_ref):
    sv = jnp.maximum(s_ref[...], 0.0)
    m_ref[...] = jnp.maximum(_dot_t(sv, w_ref[...]) + b_ref[...], 0.0)


def _node_body(x_ref, pa_ref, pb_ref, w1x_ref, w1a_ref, b1_ref, w2_ref,
               b2_ref, h_ref):
    xv = x_ref[...]
    agg = pa_ref[0] + pa_ref[1] + pb_ref[0] + pb_ref[1]
    h1 = jnp.maximum(_dot_t(xv, w1x_ref[...]) + _dot_t(agg, w1a_ref[...])
                     + b1_ref[...], 0.0)
    h_ref[...] = _dot_t(h1, w2_ref[...]) + b2_ref[...] + xv


def _gather_add_body(u_hbm, v_hbm, row_hbm, col_hbm, out_hbm,
                     ridx, cidx, ub0, vb0, ob0, ub1, vb1, ob1,
                     gs0, gs1, ws0, ws1):
    epw = ridx.shape[0]
    ch = ub0.shape[0]
    nch = epw // ch
    h = ub0.shape[1]
    wid = lax.axis_index("s") * NC + lax.axis_index("c")
    ebase = wid * epw
    pltpu.sync_copy(row_hbm.at[pl.ds(ebase, epw)], ridx)
    pltpu.sync_copy(col_hbm.at[pl.ds(ebase, epw)], cidx)

    bufs = ((ub0, vb0, ob0, gs0, ws0), (ub1, vb1, ob1, gs1, ws1))

    def fire(i, ub, vb, gs):
        off = i * ch
        pltpu.async_copy(u_hbm.at[ridx.at[pl.ds(off, ch)]], ub, gs)
        pltpu.async_copy(v_hbm.at[cidx.at[pl.ds(off, ch)]], vb, gs)

    def wait_gather(ub, vb, gs):
        pltpu.make_async_copy(u_hbm.at[ridx.at[pl.ds(0, ch)]], ub, gs).wait()
        pltpu.make_async_copy(v_hbm.at[cidx.at[pl.ds(0, ch)]], vb, gs).wait()

    def compute(ub, vb, ob):
        @plsc.parallel_loop(0, ch)
        def _row(r):
            for j in range(h // 16):
                sl = pl.ds(j * 16, 16)
                ob[r, sl] = ub[r, sl] + vb[r, sl]

    def fire_wb(i, ob, ws):
        pltpu.async_copy(ob, out_hbm.at[pl.ds(ebase + i * ch, ch)], ws)

    def wait_wb(ob, ws):
        pltpu.make_async_copy(ob, out_hbm.at[pl.ds(ebase, ch)], ws).wait()

    fire(0, ub0, vb0, gs0)
    fire(1, ub1, vb1, gs1)

    @pl.loop(0, nch // 2)
    def _main(k):
        for b in range(2):
            ub, vb, ob, gs, ws = bufs[b]
            i = 2 * k + b
            wait_gather(ub, vb, gs)

            @pl.when(k > 0)
            def _():
                wait_wb(ob, ws)

            compute(ub, vb, ob)
            fire_wb(i, ob, ws)

            @pl.when(i + 2 < nch)
            def _():
                fire(i + 2, ub, vb, gs)

    if nch % 2:
        wait_gather(ub0, vb0, gs0)
        wait_wb(ob0, ws0)
        compute(ub0, vb0, ob0)
        fire_wb(nch - 1, ob0, ws0)
    wait_wb(ob0, ws0)
    wait_wb(ob1, ws1)


def _scatter_add_body(m_hbm, row_hbm, zeros_hbm, out_hbm,
                      rb0, mb0, rb1, mb1, agg_sh, ds0, ds1):
    n = agg_sh.shape[0]
    ch = rb0.shape[0]
    epw = m_hbm.shape[0] // NW
    nch = epw // ch
    rpt = (n // NS) & ~7
    tail = n - NS * rpt
    c = lax.axis_index("c")
    sub = lax.axis_index("s")
    wid = sub * NC + c
    ebase = wid * epw
    nb = sub * rpt
    pltpu.sync_copy(zeros_hbm.at[pl.ds(nb, rpt)], agg_sh.at[pl.ds(nb, rpt)])
    if tail:
        @pl.when(sub == NS - 1)
        def _init_tail():
            pltpu.sync_copy(zeros_hbm.at[pl.ds(NS * rpt, tail)],
                            agg_sh.at[pl.ds(NS * rpt, tail)])
    plsc.subcore_barrier()

    bufs = ((rb0, mb0, ds0), (rb1, mb1, ds1))

    def fire(i, rb, mb, dsm):
        off = ebase + i * ch
        pltpu.async_copy(row_hbm.at[pl.ds(off, ch)], rb, dsm)
        pltpu.async_copy(m_hbm.at[pl.ds(off, ch)], mb, dsm)

    def wait_fire(rb, mb, dsm):
        pltpu.make_async_copy(row_hbm.at[pl.ds(ebase, ch)], rb, dsm).wait()
        pltpu.make_async_copy(m_hbm.at[pl.ds(ebase, ch)], mb, dsm).wait()

    fire(0, rb0, mb0, ds0)
    fire(1, rb1, mb1, ds1)

    @pl.loop(0, nch // 2)
    def _main(k):
        for b in range(2):
            rb, mb, dsm = bufs[b]
            i = 2 * k + b
            wait_fire(rb, mb, dsm)
            pltpu.sync_copy(mb, agg_sh.at[rb], add=True)

            @pl.when(i + 2 < nch)
            def _():
                fire(i + 2, rb, mb, dsm)

    if nch % 2:
        wait_fire(rb0, mb0, ds0)
        pltpu.sync_copy(mb0, agg_sh.at[rb0], add=True)

    plsc.subcore_barrier()
    pltpu.sync_copy(agg_sh.at[pl.ds(nb, rpt)], out_hbm.at[c, pl.ds(nb, rpt)])
    if tail:
        @pl.when(sub == NS - 1)
        def _out_tail():
            pltpu.sync_copy(agg_sh.at[pl.ds(NS * rpt, tail)],
                            out_hbm.at[c, pl.ds(NS * rpt, tail)])


NSPLIT = 2


def kernel(x, edge_index, W_e1, b_e1, W_e2, b_e2, W_n1, b_n1, W_n2, b_n2):
    n, d = x.shape
    e = edge_index.shape[1]
    h = W_e1.shape[0]
    ec = e // NSPLIT
    epw = ec // NW
    ch = 80 if epw % 80 == 0 else 40
    assert ec % NSPLIT == 0 and epw % ch == 0 and epw % 8 == 0 and n % NS == 0

    row = edge_index[0].astype(jnp.int32)
    col = edge_index[1].astype(jnp.int32)

    u, v = pl.pallas_call(
        _uv_body,
        out_shape=(jax.ShapeDtypeStruct((n, h), jnp.float32),
                   jax.ShapeDtypeStruct((n, h), jnp.float32)),
    )(x, W_e1, b_e1.reshape(1, h))

    mesh = plsc.VectorSubcoreMesh(core_axis_name="c", subcore_axis_name="s",
                                  num_cores=NC, num_subcores=NS)
    gather_call = pl.kernel(
        _gather_add_body,
        out_type=jax.ShapeDtypeStruct((ec, h), jnp.float32),
        mesh=mesh,
        scratch_types=[
            pltpu.VMEM((epw,), jnp.int32),
            pltpu.VMEM((epw,), jnp.int32),
            pltpu.VMEM((ch, h), jnp.float32),
            pltpu.VMEM((ch, h), jnp.float32),
            pltpu.VMEM((ch, h), jnp.float32),
            pltpu.VMEM((ch, h), jnp.float32),
            pltpu.VMEM((ch, h), jnp.float32),
            pltpu.VMEM((ch, h), jnp.float32),
            pltpu.SemaphoreType.DMA,
            pltpu.SemaphoreType.DMA,
            pltpu.SemaphoreType.DMA,
            pltpu.SemaphoreType.DMA,
        ],
    )
    scatter_call = pl.kernel(
        _scatter_add_body,
        out_type=jax.ShapeDtypeStruct((NC, n, h), jnp.float32),
        mesh=mesh,
        scratch_types=[
            pltpu.VMEM((ch,), jnp.int32),
            pltpu.VMEM((ch, h), jnp.float32),
            pltpu.VMEM((ch,), jnp.int32),
            pltpu.VMEM((ch, h), jnp.float32),
            pltpu.VMEM_SHARED((n, h), jnp.float32),
            pltpu.SemaphoreType.DMA,
            pltpu.SemaphoreType.DMA,
        ],
    )
    eb = 2000
    edge_call = pl.pallas_call(
        _edge_body,
        grid=(ec // eb,),
        in_specs=[
            pl.BlockSpec((eb, h), lambda i: (i, 0)),
            pl.BlockSpec((h, h), lambda i: (0, 0)),
            pl.BlockSpec((1, h), lambda i: (0, 0)),
        ],
        out_specs=pl.BlockSpec((eb, h), lambda i: (i, 0)),
        out_shape=jax.ShapeDtypeStruct((ec, h), jnp.float32),
    )

    zeros = jnp.zeros((n, h), jnp.float32)
    b_e2r = b_e2.reshape(1, h)
    m_parts, p_parts = [], []
    for c in range(NSPLIT):
        row_c = lax.slice_in_dim(row, c * ec, (c + 1) * ec)
        col_c = lax.slice_in_dim(col, c * ec, (c + 1) * ec)
        s_c = gather_call(u, v, row_c, col_c)
        m_c = edge_call(s_c, W_e2, b_e2r)
        p_c = scatter_call(m_c, row_c, zeros)
        m_parts.append(m_c)
        p_parts.append(p_c)

    m = jnp.concatenate(m_parts, axis=0)

    nb = 2000
    hout = pl.pallas_call(
        _node_body,
        grid=(n // nb,),
        in_specs=[
            pl.BlockSpec((nb, d), lambda i: (i, 0)),
            pl.BlockSpec((NC, nb, h), lambda i: (0, i, 0)),
            pl.BlockSpec((NC, nb, h), lambda i: (0, i, 0)),
            pl.BlockSpec((h, d), lambda i: (0, 0)),
            pl.BlockSpec((h, h), lambda i: (0, 0)),
            pl.BlockSpec((1, h), lambda i: (0, 0)),
            pl.BlockSpec((d, h), lambda i: (0, 0)),
            pl.BlockSpec((1, d), lambda i: (0, 0)),
        ],
        out_specs=pl.BlockSpec((nb, d), lambda i: (i, 0)),
        out_shape=jax.ShapeDtypeStruct((n, d), jnp.float32),
    )(x, p_parts[0], p_parts[1], W_n1[:, :d], W_n1[:, d:],
      b_n1.reshape(1, h), W_n2, b_n2.reshape(1, d))

    return (hout, m)


# speedup vs baseline: 1.1358x; 1.1358x over previous
import functools

import jax
import jax.numpy as jnp
from jax import lax
from jax.experimental import pallas as pl
from jax.experimental.pallas import tpu as pltpu
from jax.experimental.pallas import tpu_sc as plsc

NC = 2
NS = 16
NW = NC * NS
CH = 80


def _dot_t(a, b):
    return lax.dot_general(a, b, (((1,), (1,)), ((), ())),
                           preferred_element_type=jnp.float32)


def _uv_body(x_ref, w_ref, b_ref, u_ref, v_ref):
    d = x_ref.shape[1]
    xv = x_ref[...]
    u_ref[...] = _dot_t(xv, w_ref[:, :d]) + b_ref[...]
    v_ref[...] = _dot_t(xv, w_ref[:, d:])


def _edge_body(s_ref, w_ref, b_ref, m_ref):
    sv = jnp.maximum(s_ref[...], 0.0)
    m_ref[...] = jnp.maximum(_dot_t(sv, w_ref[...]) + b_ref[...], 0.0)


def _node_body(x_ref, pa_ref, pb_ref, w1x_ref, w1a_ref, b1_ref, w2_ref,
               b2_ref, h_ref):
    xv = x_ref[...]
    agg = pa_ref[0] + pa_ref[1] + pb_ref[0] + pb_ref[1]
    h1 = jnp.maximum(_dot_t(xv, w1x_ref[...]) + _dot_t(agg, w1a_ref[...])
                     + b1_ref[...], 0.0)
    h_ref[...] = _dot_t(h1, w2_ref[...]) + b2_ref[...] + xv


def _gather_add_body(u_hbm, v_hbm, row_hbm, col_hbm, out_hbm,
                     ridx, cidx, ub0, vb0, ob0, ub1, vb1, ob1,
                     gs0, gs1, ws0, ws1):
    epw = ridx.shape[0]
    ch = ub0.shape[0]
    nch = epw // ch
    h = ub0.shape[1]
    wid = lax.axis_index("s") * NC + lax.axis_index("c")
    ebase = wid * epw
    pltpu.sync_copy(row_hbm.at[pl.ds(ebase, epw)], ridx)
    pltpu.sync_copy(col_hbm.at[pl.ds(ebase, epw)], cidx)

    bufs = ((ub0, vb0, ob0, gs0, ws0), (ub1, vb1, ob1, gs1, ws1))

    def fire(i, ub, vb, gs):
        off = i * ch
        pltpu.async_copy(u_hbm.at[ridx.at[pl.ds(off, ch)]], ub, gs)
        pltpu.async_copy(v_hbm.at[cidx.at[pl.ds(off, ch)]], vb, gs)

    def wait_gather(ub, vb, gs):
        pltpu.make_async_copy(u_hbm.at[ridx.at[pl.ds(0, ch)]], ub, gs).wait()
        pltpu.make_async_copy(v_hbm.at[cidx.at[pl.ds(0, ch)]], vb, gs).wait()

    def compute(ub, vb, ob):
        @plsc.parallel_loop(0, ch)
        def _row(r):
            for j in range(h // 16):
                sl = pl.ds(j * 16, 16)
                ob[r, sl] = ub[r, sl] + vb[r, sl]

    def fire_wb(i, ob, ws):
        pltpu.async_copy(ob, out_hbm.at[pl.ds(ebase + i * ch, ch)], ws)

    def wait_wb(ob, ws):
        pltpu.make_async_copy(ob, out_hbm.at[pl.ds(ebase, ch)], ws).wait()

    fire(0, ub0, vb0, gs0)
    fire(1, ub1, vb1, gs1)

    @pl.loop(0, nch // 2)
    def _main(k):
        for b in range(2):
            ub, vb, ob, gs, ws = bufs[b]
            i = 2 * k + b
            wait_gather(ub, vb, gs)

            @pl.when(k > 0)
            def _():
                wait_wb(ob, ws)

            compute(ub, vb, ob)
            fire_wb(i, ob, ws)

            @pl.when(i + 2 < nch)
            def _():
                fire(i + 2, ub, vb, gs)

    if nch % 2:
        wait_gather(ub0, vb0, gs0)
        wait_wb(ob0, ws0)
        compute(ub0, vb0, ob0)
        fire_wb(nch - 1, ob0, ws0)
    wait_wb(ob0, ws0)
    wait_wb(ob1, ws1)


def _scatter_add_body(m_hbm, row_hbm, zeros_hbm, out_hbm,
                      rb0, mb0, rb1, mb1, agg_sh, ds0, ds1):
    n = agg_sh.shape[0]
    ch = rb0.shape[0]
    epw = m_hbm.shape[0] // NW
    nch = epw // ch
    rpt = (n // NS) & ~7
    tail = n - NS * rpt
    c = lax.axis_index("c")
    sub = lax.axis_index("s")
    wid = sub * NC + c
    ebase = wid * epw
    nb = sub * rpt
    pltpu.sync_copy(zeros_hbm.at[pl.ds(nb, rpt)], agg_sh.at[pl.ds(nb, rpt)])
    if tail:
        @pl.when(sub == NS - 1)
        def _init_tail():
            pltpu.sync_copy(zeros_hbm.at[pl.ds(NS * rpt, tail)],
                            agg_sh.at[pl.ds(NS * rpt, tail)])
    plsc.subcore_barrier()

    bufs = ((rb0, mb0, ds0), (rb1, mb1, ds1))

    def fire(i, rb, mb, dsm):
        off = ebase + i * ch
        pltpu.async_copy(row_hbm.at[pl.ds(off, ch)], rb, dsm)
        pltpu.async_copy(m_hbm.at[pl.ds(off, ch)], mb, dsm)

    def wait_fire(rb, mb, dsm):
        pltpu.make_async_copy(row_hbm.at[pl.ds(ebase, ch)], rb, dsm).wait()
        pltpu.make_async_copy(m_hbm.at[pl.ds(ebase, ch)], mb, dsm).wait()

    fire(0, rb0, mb0, ds0)
    fire(1, rb1, mb1, ds1)

    @pl.loop(0, nch // 2)
    def _main(k):
        for b in range(2):
            rb, mb, dsm = bufs[b]
            i = 2 * k + b
            wait_fire(rb, mb, dsm)
            pltpu.sync_copy(mb, agg_sh.at[rb], add=True)

            @pl.when(i + 2 < nch)
            def _():
                fire(i + 2, rb, mb, dsm)

    if nch % 2:
        wait_fire(rb0, mb0, ds0)
        pltpu.sync_copy(mb0, agg_sh.at[rb0], add=True)

    plsc.subcore_barrier()
    pltpu.sync_copy(agg_sh.at[pl.ds(nb, rpt)], out_hbm.at[c, pl.ds(nb, rpt)])
    if tail:
        @pl.when(sub == NS - 1)
        def _out_tail():
            pltpu.sync_copy(agg_sh.at[pl.ds(NS * rpt, tail)],
                            out_hbm.at[c, pl.ds(NS * rpt, tail)])


CH = 80


def kernel(x, edge_index, W_e1, b_e1, W_e2, b_e2, W_n1, b_n1, W_n2, b_n2):
    n, d = x.shape
    e = edge_index.shape[1]
    h = W_e1.shape[0]

    grain = NW * CH
    ec0 = (e // (2 * grain)) * grain
    ecs = [ec0, e - ec0]
    assert all(c > 0 and c % grain == 0 for c in ecs) and n % NS == 0

    row = edge_index[0].astype(jnp.int32)
    col = edge_index[1].astype(jnp.int32)

    u, v = pl.pallas_call(
        _uv_body,
        out_shape=(jax.ShapeDtypeStruct((n, h), jnp.float32),
                   jax.ShapeDtypeStruct((n, h), jnp.float32)),
    )(x, W_e1, b_e1.reshape(1, h))

    mesh = plsc.VectorSubcoreMesh(core_axis_name="c", subcore_axis_name="s",
                                  num_cores=NC, num_subcores=NS)

    zeros = jnp.zeros((n, h), jnp.float32)
    b_e2r = b_e2.reshape(1, h)
    m_parts, p_parts = [], []
    off = 0
    for ec in ecs:
        epw = ec // NW
        gather_call = pl.kernel(
            _gather_add_body,
            out_type=jax.ShapeDtypeStruct((ec, h), jnp.float32),
            mesh=mesh,
            scratch_types=[
                pltpu.VMEM((epw,), jnp.int32),
                pltpu.VMEM((epw,), jnp.int32),
                pltpu.VMEM((CH, h), jnp.float32),
                pltpu.VMEM((CH, h), jnp.float32),
                pltpu.VMEM((CH, h), jnp.float32),
                pltpu.VMEM((CH, h), jnp.float32),
                pltpu.VMEM((CH, h), jnp.float32),
                pltpu.VMEM((CH, h), jnp.float32),
                pltpu.SemaphoreType.DMA,
                pltpu.SemaphoreType.DMA,
                pltpu.SemaphoreType.DMA,
                pltpu.SemaphoreType.DMA,
            ],
        )
        scatter_call = pl.kernel(
            _scatter_add_body,
            out_type=jax.ShapeDtypeStruct((NC, n, h), jnp.float32),
            mesh=mesh,
            scratch_types=[
                pltpu.VMEM((CH,), jnp.int32),
                pltpu.VMEM((CH, h), jnp.float32),
                pltpu.VMEM((CH,), jnp.int32),
                pltpu.VMEM((CH, h), jnp.float32),
                pltpu.VMEM_SHARED((n, h), jnp.float32),
                pltpu.SemaphoreType.DMA,
                pltpu.SemaphoreType.DMA,
            ],
        )
        eb = ec // 80
        assert ec % eb == 0 and eb % 8 == 0
        edge_call = pl.pallas_call(
            _edge_body,
            grid=(ec // eb,),
            in_specs=[
                pl.BlockSpec((eb, h), lambda i: (i, 0)),
                pl.BlockSpec((h, h), lambda i: (0, 0)),
                pl.BlockSpec((1, h), lambda i: (0, 0)),
            ],
            out_specs=pl.BlockSpec((eb, h), lambda i: (i, 0)),
            out_shape=jax.ShapeDtypeStruct((ec, h), jnp.float32),
        )

        row_c = lax.slice_in_dim(row, off, off + ec)
        col_c = lax.slice_in_dim(col, off, off + ec)
        s_c = gather_call(u, v, row_c, col_c)
        m_c = edge_call(s_c, W_e2, b_e2r)
        p_c = scatter_call(m_c, row_c, zeros)
        m_parts.append(m_c)
        p_parts.append(p_c)
        off += ec

    m = jnp.concatenate(m_parts, axis=0)

    nb = 2000
    hout = pl.pallas_call(
        _node_body,
        grid=(n // nb,),
        in_specs=[
            pl.BlockSpec((nb, d), lambda i: (i, 0)),
            pl.BlockSpec((NC, nb, h), lambda i: (0, i, 0)),
            pl.BlockSpec((NC, nb, h), lambda i: (0, i, 0)),
            pl.BlockSpec((h, d), lambda i: (0, 0)),
            pl.BlockSpec((h, h), lambda i: (0, 0)),
            pl.BlockSpec((1, h), lambda i: (0, 0)),
            pl.BlockSpec((d, h), lambda i: (0, 0)),
            pl.BlockSpec((1, d), lambda i: (0, 0)),
        ],
        out_specs=pl.BlockSpec((nb, d), lambda i: (i, 0)),
        out_shape=jax.ShapeDtypeStruct((n, d), jnp.float32),
    )(x, p_parts[0], p_parts[1], W_n1[:, :d], W_n1[:, d:],
      b_n1.reshape(1, h), W_n2, b_n2.reshape(1, d))

    return (hout, m)


# speedup vs baseline: 1.1952x; 1.0524x over previous
import functools

import jax
import jax.numpy as jnp
from jax import lax
from jax.experimental import pallas as pl
from jax.experimental.pallas import tpu as pltpu
from jax.experimental.pallas import tpu_sc as plsc

NC = 2
NS = 16
NW = NC * NS
CH = 80


def _dot_t(a, b):
    return lax.dot_general(a, b, (((1,), (1,)), ((), ())),
                           preferred_element_type=jnp.float32)


def _uv_body(x_ref, w_ref, b_ref, u_ref, v_ref):
    d = x_ref.shape[1]
    xv = x_ref[...]
    u_ref[...] = _dot_t(xv, w_ref[:, :d]) + b_ref[...]
    v_ref[...] = _dot_t(xv, w_ref[:, d:])


def _edge_body(s_ref, w_ref, b_ref, mc_ref, mf_ref):
    sv = jnp.maximum(s_ref[...], 0.0)
    val = jnp.maximum(_dot_t(sv, w_ref[...]) + b_ref[...], 0.0)
    mc_ref[...] = val
    mf_ref[...] = val


def _edge_body_acc(s_ref, w_ref, b_ref, macc_ref, mc_ref, mf_ref):
    del macc_ref
    _edge_body(s_ref, w_ref, b_ref, mc_ref, mf_ref)


def _node_body(x_ref, p_ref, w1x_ref, w1a_ref, b1_ref, w2_ref,
               b2_ref, h_ref):
    xv = x_ref[...]
    agg = p_ref[0] + p_ref[1]
    h1 = jnp.maximum(_dot_t(xv, w1x_ref[...]) + _dot_t(agg, w1a_ref[...])
                     + b1_ref[...], 0.0)
    h_ref[...] = _dot_t(h1, w2_ref[...]) + b2_ref[...] + xv


def _gather_add_body(u_hbm, v_hbm, row_hbm, col_hbm, out_hbm,
                     ridx, cidx, ub0, vb0, ob0, ub1, vb1, ob1,
                     gs0, gs1, ws0, ws1):
    epw = ridx.shape[0]
    ch = ub0.shape[0]
    nch = epw // ch
    h = ub0.shape[1]
    wid = lax.axis_index("s") * NC + lax.axis_index("c")
    ebase = wid * epw
    pltpu.sync_copy(row_hbm.at[pl.ds(ebase, epw)], ridx)
    pltpu.sync_copy(col_hbm.at[pl.ds(ebase, epw)], cidx)

    bufs = ((ub0, vb0, ob0, gs0, ws0), (ub1, vb1, ob1, gs1, ws1))

    def fire(i, ub, vb, gs):
        off = i * ch
        pltpu.async_copy(u_hbm.at[ridx.at[pl.ds(off, ch)]], ub, gs)
        pltpu.async_copy(v_hbm.at[cidx.at[pl.ds(off, ch)]], vb, gs)

    def wait_gather(ub, vb, gs):
        pltpu.make_async_copy(u_hbm.at[ridx.at[pl.ds(0, ch)]], ub, gs).wait()
        pltpu.make_async_copy(v_hbm.at[cidx.at[pl.ds(0, ch)]], vb, gs).wait()

    def compute(ub, vb, ob):
        @plsc.parallel_loop(0, ch)
        def _row(r):
            for j in range(h // 16):
                sl = pl.ds(j * 16, 16)
                ob[r, sl] = ub[r, sl] + vb[r, sl]

    def fire_wb(i, ob, ws):
        pltpu.async_copy(ob, out_hbm.at[pl.ds(ebase + i * ch, ch)], ws)

    def wait_wb(ob, ws):
        pltpu.make_async_copy(ob, out_hbm.at[pl.ds(ebase, ch)], ws).wait()

    fire(0, ub0, vb0, gs0)
    fire(1, ub1, vb1, gs1)

    @pl.loop(0, nch // 2)
    def _main(k):
        for b in range(2):
            ub, vb, ob, gs, ws = bufs[b]
            i = 2 * k + b
            wait_gather(ub, vb, gs)

            @pl.when(k > 0)
            def _():
                wait_wb(ob, ws)

            compute(ub, vb, ob)
            fire_wb(i, ob, ws)

            @pl.when(i + 2 < nch)
            def _():
                fire(i + 2, ub, vb, gs)

    if nch % 2:
        wait_gather(ub0, vb0, gs0)
        wait_wb(ob0, ws0)
        compute(ub0, vb0, ob0)
        fire_wb(nch - 1, ob0, ws0)
    wait_wb(ob0, ws0)
    wait_wb(ob1, ws1)


def _scatter_add_body(m_hbm, row_hbm, init_hbm, out_hbm,
                      rb0, mb0, rb1, mb1, agg_sh, ds0, ds1):
    n = agg_sh.shape[0]
    ch = rb0.shape[0]
    epw = m_hbm.shape[0] // NW
    nch = epw // ch
    rpt = (n // NS) & ~7
    tail = n - NS * rpt
    c = lax.axis_index("c")
    sub = lax.axis_index("s")
    wid = sub * NC + c
    ebase = wid * epw
    nb = sub * rpt
    pltpu.sync_copy(init_hbm.at[c, pl.ds(nb, rpt)], agg_sh.at[pl.ds(nb, rpt)])
    if tail:
        @pl.when(sub == NS - 1)
        def _init_tail():
            pltpu.sync_copy(init_hbm.at[c, pl.ds(NS * rpt, tail)],
                            agg_sh.at[pl.ds(NS * rpt, tail)])
    plsc.subcore_barrier()

    bufs = ((rb0, mb0, ds0), (rb1, mb1, ds1))

    def fire(i, rb, mb, dsm):
        off = ebase + i * ch
        pltpu.async_copy(row_hbm.at[pl.ds(off, ch)], rb, dsm)
        pltpu.async_copy(m_hbm.at[pl.ds(off, ch)], mb, dsm)

    def wait_fire(rb, mb, dsm):
        pltpu.make_async_copy(row_hbm.at[pl.ds(ebase, ch)], rb, dsm).wait()
        pltpu.make_async_copy(m_hbm.at[pl.ds(ebase, ch)], mb, dsm).wait()

    fire(0, rb0, mb0, ds0)
    fire(1, rb1, mb1, ds1)

    @pl.loop(0, nch // 2)
    def _main(k):
        for b in range(2):
            rb, mb, dsm = bufs[b]
            i = 2 * k + b
            wait_fire(rb, mb, dsm)
            pltpu.sync_copy(mb, agg_sh.at[rb], add=True)

            @pl.when(i + 2 < nch)
            def _():
                fire(i + 2, rb, mb, dsm)

    if nch % 2:
        wait_fire(rb0, mb0, ds0)
        pltpu.sync_copy(mb0, agg_sh.at[rb0], add=True)

    plsc.subcore_barrier()
    pltpu.sync_copy(agg_sh.at[pl.ds(nb, rpt)], out_hbm.at[c, pl.ds(nb, rpt)])
    if tail:
        @pl.when(sub == NS - 1)
        def _out_tail():
            pltpu.sync_copy(agg_sh.at[pl.ds(NS * rpt, tail)],
                            out_hbm.at[c, pl.ds(NS * rpt, tail)])


def kernel(x, edge_index, W_e1, b_e1, W_e2, b_e2, W_n1, b_n1, W_n2, b_n2):
    n, d = x.shape
    e = edge_index.shape[1]
    h = W_e1.shape[0]

    grain = NW * CH
    ec0 = (e // (2 * grain)) * grain
    ecs = [ec0, e - ec0]
    assert all(c > 0 and c % grain == 0 for c in ecs) and n % NS == 0

    row = edge_index[0].astype(jnp.int32)
    col = edge_index[1].astype(jnp.int32)

    u, v = pl.pallas_call(
        _uv_body,
        out_shape=(jax.ShapeDtypeStruct((n, h), jnp.float32),
                   jax.ShapeDtypeStruct((n, h), jnp.float32)),
    )(x, W_e1, b_e1.reshape(1, h))

    mesh = plsc.VectorSubcoreMesh(core_axis_name="c", subcore_axis_name="s",
                                  num_cores=NC, num_subcores=NS)

    zeros = jnp.zeros((NC, n, h), jnp.float32)
    b_e2r = b_e2.reshape(1, h)
    m_acc = None
    p_acc = zeros
    off = 0
    for ec in ecs:
        epw = ec // NW
        gather_call = pl.kernel(
            _gather_add_body,
            out_type=jax.ShapeDtypeStruct((ec, h), jnp.float32),
            mesh=mesh,
            scratch_types=[
                pltpu.VMEM((epw,), jnp.int32),
                pltpu.VMEM((epw,), jnp.int32),
                pltpu.VMEM((CH, h), jnp.float32),
                pltpu.VMEM((CH, h), jnp.float32),
                pltpu.VMEM((CH, h), jnp.float32),
                pltpu.VMEM((CH, h), jnp.float32),
                pltpu.VMEM((CH, h), jnp.float32),
                pltpu.VMEM((CH, h), jnp.float32),
                pltpu.SemaphoreType.DMA,
                pltpu.SemaphoreType.DMA,
                pltpu.SemaphoreType.DMA,
                pltpu.SemaphoreType.DMA,
            ],
        )
        scatter_call = pl.kernel(
            _scatter_add_body,
            out_type=jax.ShapeDtypeStruct((NC, n, h), jnp.float32),
            mesh=mesh,
            scratch_types=[
                pltpu.VMEM((CH,), jnp.int32),
                pltpu.VMEM((CH, h), jnp.float32),
                pltpu.VMEM((CH,), jnp.int32),
                pltpu.VMEM((CH, h), jnp.float32),
                pltpu.VMEM_SHARED((n, h), jnp.float32),
                pltpu.SemaphoreType.DMA,
                pltpu.SemaphoreType.DMA,
            ],
        )
        eb = grain
        assert ec % eb == 0 and off % eb == 0
        o = off // eb
        base_in_specs = [
            pl.BlockSpec((eb, h), lambda i: (i, 0)),
            pl.BlockSpec((h, h), lambda i: (0, 0)),
            pl.BlockSpec((1, h), lambda i: (0, 0)),
        ]
        out_specs = [
            pl.BlockSpec((eb, h), lambda i: (i, 0)),
            pl.BlockSpec((eb, h), lambda i, o=o: (i + o, 0)),
        ]
        out_shape = [jax.ShapeDtypeStruct((ec, h), jnp.float32),
                     jax.ShapeDtypeStruct((e, h), jnp.float32)]
        if m_acc is None:
            edge_call = pl.pallas_call(
                _edge_body, grid=(ec // eb,), in_specs=base_in_specs,
                out_specs=out_specs, out_shape=out_shape)
        else:
            edge_call = pl.pallas_call(
                _edge_body_acc, grid=(ec // eb,),
                in_specs=base_in_specs + [
                    pl.BlockSpec(memory_space=pltpu.MemorySpace.HBM)],
                out_specs=out_specs, out_shape=out_shape,
                input_output_aliases={3: 1})

        row_c = lax.slice_in_dim(row, off, off + ec)
        col_c = lax.slice_in_dim(col, off, off + ec)
        s_c = gather_call(u, v, row_c, col_c)
        if m_acc is None:
            m_c, m_acc = edge_call(s_c, W_e2, b_e2r)
        else:
            m_c, m_acc = edge_call(s_c, W_e2, b_e2r, m_acc)
        p_acc = scatter_call(m_c, row_c, p_acc)
        off += ec

    m = m_acc

    nb = 2000
    hout = pl.pallas_call(
        _node_body,
        grid=(n // nb,),
        in_specs=[
            pl.BlockSpec((nb, d), lambda i: (i, 0)),
            pl.BlockSpec((NC, nb, h), lambda i: (0, i, 0)),
            pl.BlockSpec((h, d), lambda i: (0, 0)),
            pl.BlockSpec((h, h), lambda i: (0, 0)),
            pl.BlockSpec((1, h), lambda i: (0, 0)),
            pl.BlockSpec((d, h), lambda i: (0, 0)),
            pl.BlockSpec((1, d), lambda i: (0, 0)),
        ],
        out_specs=pl.BlockSpec((nb, d), lambda i: (i, 0)),
        out_shape=jax.ShapeDtypeStruct((n, d), jnp.float32),
    )(x, p_acc, W_n1[:, :d], W_n1[:, d:],
      b_n1.reshape(1, h), W_n2, b_n2.reshape(1, d))

    return (hout, m)
